# baseline (speedup 1.0000x reference)
import jax
import jax.numpy as jnp
from jax import lax
from jax.experimental import pallas as pl
from jax.experimental.pallas import tpu as pltpu
from jax.experimental.pallas import tpu_sc as plsc

N = 10000
E = 320000
D = 128
NDIMS = 3

NC = 2
NS = 16
NW = NC * NS
OWN = 320
ACC_ROWS = 328
OUT_ROWS = NW * OWN

ED = 2 * E
CHUNK = 128
GIDX = 5120
NGRP = ED // GIDX
FLUSH = 2048
SBUF = 8192
CAPB = (ED // FLUSH) + 2
CAP = CAPB * FLUSH


def _sc_bucketize_kernel(srcg_hbm, dstg_hbm, lsrc_hbm, ldst_hbm, cnt_hbm,
                         gsrc_v, gdst_v, csrc_v, cdst_v, pbuf_v, tmp_v,
                         ig0, ig1):
    c = lax.axis_index("c")
    s = lax.axis_index("s")
    w = c * NS + s
    row0 = w * OWN
    win0 = s * ACC_ROWS
    lane = lax.iota(jnp.int32, 16)
    pad_src = lane & 7
    pad_dst = win0 + OWN + (lane & 7)

    g0 = lax.rem(w * (NGRP // NW), NGRP)

    def goff(gi):
        return lax.rem(g0 + gi, NGRP) * GIDX

    def issue_group(gi, b):
        sem = ig0 if b == 0 else ig1
        pltpu.async_copy(srcg_hbm.at[pl.ds(goff(gi), GIDX)],
                         gsrc_v.at[pl.ds(b * GIDX, GIDX)], sem)
        pltpu.async_copy(dstg_hbm.at[pl.ds(goff(gi), GIDX)],
                         gdst_v.at[pl.ds(b * GIDX, GIDX)], sem)

    def wait_group(gi, b):
        sem = ig0 if b == 0 else ig1
        pltpu.make_async_copy(srcg_hbm.at[pl.ds(goff(gi), GIDX)],
                              gsrc_v.at[pl.ds(b * GIDX, GIDX)], sem).wait()
        pltpu.make_async_copy(dstg_hbm.at[pl.ds(goff(gi), GIDX)],
                              gdst_v.at[pl.ds(b * GIDX, GIDX)], sem).wait()

    issue_group(0, 0)

    def flush_blocks(n_new, nf):
        def fl(i, nf2):
            sb = lax.rem(nf2 * FLUSH, SBUF)
            pltpu.sync_copy(csrc_v.at[pl.ds(sb, FLUSH)],
                            lsrc_hbm.at[w, pl.ds(nf2 * FLUSH, FLUSH)])
            pltpu.sync_copy(cdst_v.at[pl.ds(sb, FLUSH)],
                            ldst_hbm.at[w, pl.ds(nf2 * FLUSH, FLUSH)])
            return nf2 + 1

        return lax.fori_loop(0, n_new, fl, nf)

    def group_body(gi, carry):
        off, nf = carry
        bb = lax.rem(gi, 2)

        @pl.when((gi + 1 < NGRP) & (bb == 0))
        def _():
            issue_group(gi + 1, 1)

        @pl.when((gi + 1 < NGRP) & (bb == 1))
        def _():
            issue_group(gi + 1, 0)

        @pl.when(bb == 0)
        def _():
            wait_group(gi, 0)

        @pl.when(bb == 1)
        def _():
            wait_group(gi, 1)

        base = bb * GIDX

        def pass_a1(k, z):
            d16 = gdst_v[pl.ds(base + k * 16, 16)] - row0
            m = (d16 >= 0) & (d16 < OWN)
            pbuf_v[pl.ds(k * 16, 16)] = plsc.all_reduce_population_count(m)
            return z

        lax.fori_loop(0, GIDX // 16, pass_a1, 0)

        def pass_a2(k, off_a):
            v = pbuf_v[pl.ds(k * 16, 16)]
            pbuf_v[pl.ds(k * 16, 16)] = jnp.broadcast_to(off_a, (16,))
            return off_a + v[0]

        off_end = lax.fori_loop(0, GIDX // 16, pass_a2, off)

        def pass_b(k, z):
            d16 = gdst_v[pl.ds(base + k * 16, 16)] - row0
            s16 = gsrc_v[pl.ds(base + k * 16, 16)]
            m = (d16 >= 0) & (d16 < OWN)
            mi = jnp.where(m, jnp.full((16,), 1, jnp.int32),
                           jnp.full((16,), 0, jnp.int32))
            cs = plsc.cumsum(mi)
            pv = pbuf_v[pl.ds(k * 16, 16)]
            pos = jnp.where(m, (pv + cs - 1) & (SBUF - 1), SBUF + lane)
            plsc.store_scatter(cdst_v, [pos], d16 + win0)
            plsc.store_scatter(csrc_v, [pos], s16)
            return z

        lax.fori_loop(0, GIDX // 16, pass_b, 0)

        nf = flush_blocks(off_end // FLUSH - nf, nf)
        return off_end, nf

    off, nf = lax.fori_loop(0, NGRP, group_body,
                            (jnp.int32(0), jnp.int32(0)))

    npadv = (FLUSH - lax.rem(off, FLUSH) + 15) // 16

    def pad_body(i, off3):
        pos = (off3 + lane) & (SBUF - 1)
        plsc.store_scatter(csrc_v, [pos], pad_src)
        plsc.store_scatter(cdst_v, [pos], pad_dst)
        return off3 + 16

    off = lax.fori_loop(0, npadv, pad_body, off)
    nf = flush_blocks(off // FLUSH - nf, nf)

    tmp_v[pl.ds(0, 16)] = jnp.broadcast_to(nf, (16,))
    pltpu.sync_copy(tmp_v, cnt_hbm.at[w])


def _sc_bucketize(srcg, dstg):
    mesh = plsc.VectorSubcoreMesh(core_axis_name="c", subcore_axis_name="s")
    return pl.kernel(
        _sc_bucketize_kernel,
        out_type=(
            jax.ShapeDtypeStruct((NW, CAP), jnp.int32),
            jax.ShapeDtypeStruct((NW, CAP), jnp.int32),
            jax.ShapeDtypeStruct((NW, 16), jnp.int32),
        ),
        mesh=mesh,
        compiler_params=pltpu.CompilerParams(needs_layout_passes=False),
        scratch_types=[
            pltpu.VMEM((2 * GIDX,), jnp.int32),
            pltpu.VMEM((2 * GIDX,), jnp.int32),
            pltpu.VMEM((SBUF + 16,), jnp.int32),
            pltpu.VMEM((SBUF + 16,), jnp.int32),
            pltpu.VMEM((GIDX,), jnp.int32),
            pltpu.VMEM((16,), jnp.int32),
            pltpu.SemaphoreType.DMA,
            pltpu.SemaphoreType.DMA,
        ],
    )(srcg, dstg)


def _sc_agg_kernel(x_hbm, lsrc_hbm, ldst_hbm, cnt_hbm, out_hbm,
                   sidx_v, didx_v, rows_a, rows_b, zbuf_v, tmp_v, acc_sh,
                   g0, g1, s0, s1):
    c = lax.axis_index("c")
    s = lax.axis_index("s")
    w = c * NS + s
    win0 = s * ACC_ROWS

    zeros16 = jnp.zeros((16,), jnp.float32)

    def zrow(i, carry):
        for j in range(D // 16):
            zbuf_v[i, pl.ds(j * 16, 16)] = zeros16
        return carry

    lax.fori_loop(0, ACC_ROWS, zrow, 0)

    pltpu.sync_copy(cnt_hbm.at[w], tmp_v)
    nch = tmp_v[pl.ds(0, 16)][0] * (FLUSH // CHUNK)

    pltpu.sync_copy(zbuf_v, acc_sh.at[pl.ds(win0, ACC_ROWS)])

    def idx_dma(ci, slot):
        pltpu.sync_copy(lsrc_hbm.at[w, pl.ds(ci * CHUNK, CHUNK)],
                        sidx_v.at[slot])
        pltpu.sync_copy(ldst_hbm.at[w, pl.ds(ci * CHUNK, CHUNK)],
                        didx_v.at[slot])

    @pl.when(nch > 0)
    def _():
        idx_dma(0, 0)
        pltpu.async_copy(x_hbm.at[sidx_v.at[0]], rows_a, g0)

    @pl.when(nch > 1)
    def _():
        idx_dma(1, 1)

    def chunk_body(ci, carry):
        b = lax.rem(ci, 2)
        q = lax.rem(ci, 4)
        qn = lax.rem(ci + 1, 4)
        qp = lax.rem(ci + 2, 4)

        @pl.when(b == 0)
        def _():
            pltpu.make_async_copy(x_hbm.at[sidx_v.at[q]], rows_a, g0).wait()

            @pl.when(ci >= 1)
            def _():
                pltpu.make_async_copy(rows_b, acc_sh.at[didx_v.at[0]],
                                      s1).wait()

            @pl.when(ci + 1 < nch)
            def _():
                pltpu.async_copy(x_hbm.at[sidx_v.at[qn]], rows_b, g1)
            pltpu.async_copy(rows_a, acc_sh.at[didx_v.at[q]], s0, add=True)

            @pl.when(ci + 2 < nch)
            def _():
                idx_dma(ci + 2, qp)

        @pl.when(b == 1)
        def _():
            pltpu.make_async_copy(x_hbm.at[sidx_v.at[q]], rows_b, g1).wait()
            pltpu.make_async_copy(rows_a, acc_sh.at[didx_v.at[0]], s0).wait()

            @pl.when(ci + 1 < nch)
            def _():
                pltpu.async_copy(x_hbm.at[sidx_v.at[qn]], rows_a, g0)
            pltpu.async_copy(rows_b, acc_sh.at[didx_v.at[q]], s1, add=True)

            @pl.when(ci + 2 < nch)
            def _():
                idx_dma(ci + 2, qp)

        return carry

    lax.fori_loop(0, nch, chunk_body, 0)

    @pl.when(nch > 0)
    def _():
        @pl.when(lax.rem(nch - 1, 2) == 0)
        def _():
            pltpu.make_async_copy(rows_a, acc_sh.at[didx_v.at[0]],
                                  s0).wait()

        @pl.when(lax.rem(nch - 1, 2) == 1)
        def _():
            pltpu.make_async_copy(rows_b, acc_sh.at[didx_v.at[0]],
                                  s1).wait()

    pltpu.sync_copy(acc_sh.at[pl.ds(win0, OWN)],
                    out_hbm.at[pl.ds(w * OWN, OWN)])


def _sc_agg(x, lsrc, ldst, cnt):
    mesh = plsc.VectorSubcoreMesh(core_axis_name="c", subcore_axis_name="s")
    return pl.kernel(
        _sc_agg_kernel,
        out_type=jax.ShapeDtypeStruct((OUT_ROWS, D), jnp.float32),
        mesh=mesh,
        scratch_types=[
            pltpu.VMEM((4, CHUNK), jnp.int32),
            pltpu.VMEM((4, CHUNK), jnp.int32),
            pltpu.VMEM((CHUNK, D), jnp.float32),
            pltpu.VMEM((CHUNK, D), jnp.float32),
            pltpu.VMEM((ACC_ROWS, D), jnp.float32),
            pltpu.VMEM((16,), jnp.int32),
            pltpu.VMEM_SHARED((NS * ACC_ROWS, D), jnp.float32),
            pltpu.SemaphoreType.DMA,
            pltpu.SemaphoreType.DMA,
            pltpu.SemaphoreType.DMA,
            pltpu.SemaphoreType.DMA,
        ],
    )(x, lsrc, ldst, cnt)


BLK = 1000


def _dense_core(x_ref, a_ref, w0_ref, w1_ref, b_ref):
    y = lax.dot_general(x_ref[...], w0_ref[...], (((1,), (0,)), ((), ())),
                        preferred_element_type=jnp.float32)
    y = y + lax.dot_general(a_ref[...], w1_ref[...], (((1,), (0,)), ((), ())),
                            preferred_element_type=jnp.float32)
    return y + b_ref[...]


def _dense_body_ln(x_ref, a_ref, w0_ref, w1_ref, b_ref, o_ref):
    y = _dense_core(x_ref, a_ref, w0_ref, w1_ref, b_ref)
    m = jnp.mean(y, axis=1, keepdims=True)
    v = jnp.mean(jnp.square(y - m), axis=1, keepdims=True)
    y = (y - m) * lax.rsqrt(v + 1e-5)
    o_ref[...] = jnp.maximum(y, 0.0)


def _dense_body_ln_res(x_ref, a_ref, w0_ref, w1_ref, b_ref, r_ref, o_ref):
    y = _dense_core(x_ref, a_ref, w0_ref, w1_ref, b_ref)
    m = jnp.mean(y, axis=1, keepdims=True)
    v = jnp.mean(jnp.square(y - m), axis=1, keepdims=True)
    y = (y - m) * lax.rsqrt(v + 1e-5)
    o_ref[...] = jnp.maximum(y, 0.0) + r_ref[...]


def _dense_body_plain(x_ref, a_ref, w0_ref, w1_ref, b_ref, o_ref):
    o_ref[...] = _dense_core(x_ref, a_ref, w0_ref, w1_ref, b_ref)


def _dense(x, agg, w0, w1, b, res=None, ln_relu=True):
    grid = (N // BLK,)
    in_specs = [
        pl.BlockSpec((BLK, D), lambda i: (i, 0)),
        pl.BlockSpec((BLK, D), lambda i: (i, 0)),
        pl.BlockSpec((D, D), lambda i: (0, 0)),
        pl.BlockSpec((D, D), lambda i: (0, 0)),
        pl.BlockSpec((1, D), lambda i: (0, 0)),
    ]
    args = [x, agg, w0, w1, b]
    if res is not None:
        body = _dense_body_ln_res
        in_specs.append(pl.BlockSpec((BLK, D), lambda i: (i, 0)))
        args.append(res)
    elif ln_relu:
        body = _dense_body_ln
    else:
        body = _dense_body_plain
    return pl.pallas_call(
        body,
        grid=grid,
        in_specs=in_specs,
        out_specs=pl.BlockSpec((BLK, D), lambda i: (i, 0)),
        out_shape=jax.ShapeDtypeStruct((N, D), jnp.float32),
    )(*args)


def _stage(x):
    return pl.pallas_call(
        lambda i_ref, o_ref: o_ref.__setitem__((...,), i_ref[...]),
        grid=(N // BLK,),
        in_specs=[pl.BlockSpec((BLK, D), lambda i: (i, 0))],
        out_specs=pl.BlockSpec((BLK, D), lambda i: (i, 0)),
        out_shape=jax.ShapeDtypeStruct((N, D), jnp.float32),
    )(x)


def kernel(features, edges, w0_in, w1_in, b_in, w0_h1, w1_h1, b_h1,
           w0_h2, w1_h2, b_h2, w0_out, w1_out, b_out):
    srcg = jnp.concatenate([edges[:, 0], edges[:, 1]])
    dstg = jnp.concatenate([edges[:, 1], edges[:, 0]])

    lsrc, ldst, cnt = _sc_bucketize(srcg, dstg)

    b_in2 = b_in.reshape(1, -1)
    b_h12 = b_h1.reshape(1, -1)
    b_h22 = b_h2.reshape(1, -1)
    w0o = jnp.zeros((D, D), jnp.float32).at[:, :NDIMS].set(w0_out)
    w1o = jnp.zeros((D, D), jnp.float32).at[:, :NDIMS].set(w1_out)
    bo = jnp.zeros((1, D), jnp.float32).at[0, :NDIMS].set(b_out)

    a = _sc_agg(_stage(features), lsrc, ldst, cnt)
    h0 = _dense(features, a, w0_in, w1_in, b_in2)
    a = _sc_agg(_stage(h0), lsrc, ldst, cnt)
    h1 = _dense(h0, a, w0_h1, w1_h1, b_h12)
    a = _sc_agg(_stage(h1), lsrc, ldst, cnt)
    latent = _dense(h1, a, w0_h2, w1_h2, b_h22, res=h0)
    a = _sc_agg(_stage(latent), lsrc, ldst, cnt)
    out = _dense(latent, a, w0o, w1o, bo, ln_relu=False)
    return out[:, :NDIMS].reshape(1, 1, N, NDIMS)


# speedup vs baseline: 1.0963x; 1.0963x over previous
import jax
import jax.numpy as jnp
from jax import lax
from jax.experimental import pallas as pl
from jax.experimental.pallas import tpu as pltpu
from jax.experimental.pallas import tpu_sc as plsc

N = 10000
E = 320000
D = 128
NDIMS = 3

NC = 2
NS = 16
NW = NC * NS
OWN = 320
ACC_ROWS = 328
OUT_ROWS = NW * OWN

ED = 2 * E
CHUNK = 128
GIDX = 5120
NGRP = ED // GIDX
FLUSH = 2048
SBUF = 8192
CAPB = (ED // FLUSH) + 2
CAP = CAPB * FLUSH


def _sc_bucketize_kernel(srcg_hbm, dstg_hbm, lsrc_hbm, ldst_hbm, cnt_hbm,
                         gsrc_v, gdst_v, csrc_v, cdst_v, pbuf_v, tmp_v,
                         ig0, ig1):
    c = lax.axis_index("c")
    s = lax.axis_index("s")
    w = c * NS + s
    row0 = w * OWN
    win0 = s * ACC_ROWS
    lane = lax.iota(jnp.int32, 16)
    pad_src = lane & 7
    pad_dst = win0 + OWN + (lane & 7)

    g0 = lax.rem(w * (NGRP // NW), NGRP)

    def goff(gi):
        return lax.rem(g0 + gi, NGRP) * GIDX

    def issue_group(gi, b):
        sem = ig0 if b == 0 else ig1
        pltpu.async_copy(srcg_hbm.at[pl.ds(goff(gi), GIDX)],
                         gsrc_v.at[pl.ds(b * GIDX, GIDX)], sem)
        pltpu.async_copy(dstg_hbm.at[pl.ds(goff(gi), GIDX)],
                         gdst_v.at[pl.ds(b * GIDX, GIDX)], sem)

    def wait_group(gi, b):
        sem = ig0 if b == 0 else ig1
        pltpu.make_async_copy(srcg_hbm.at[pl.ds(goff(gi), GIDX)],
                              gsrc_v.at[pl.ds(b * GIDX, GIDX)], sem).wait()
        pltpu.make_async_copy(dstg_hbm.at[pl.ds(goff(gi), GIDX)],
                              gdst_v.at[pl.ds(b * GIDX, GIDX)], sem).wait()

    issue_group(0, 0)

    def flush_blocks(n_new, nf):
        def fl(i, nf2):
            sb = lax.rem(nf2 * FLUSH, SBUF)
            pltpu.sync_copy(csrc_v.at[pl.ds(sb, FLUSH)],
                            lsrc_hbm.at[w, pl.ds(nf2 * FLUSH, FLUSH)])
            pltpu.sync_copy(cdst_v.at[pl.ds(sb, FLUSH)],
                            ldst_hbm.at[w, pl.ds(nf2 * FLUSH, FLUSH)])
            return nf2 + 1

        return lax.fori_loop(0, n_new, fl, nf)

    def group_body(gi, carry):
        off, nf = carry
        bb = lax.rem(gi, 2)

        @pl.when((gi + 1 < NGRP) & (bb == 0))
        def _():
            issue_group(gi + 1, 1)

        @pl.when((gi + 1 < NGRP) & (bb == 1))
        def _():
            issue_group(gi + 1, 0)

        @pl.when(bb == 0)
        def _():
            wait_group(gi, 0)

        @pl.when(bb == 1)
        def _():
            wait_group(gi, 1)

        base = bb * GIDX

        def pass_a(k, off_a):
            d16 = gdst_v[pl.ds(base + k * 16, 16)] - row0
            m = (d16 >= 0) & (d16 < OWN)
            cntv = plsc.all_reduce_population_count(m)
            pbuf_v[pl.ds(k * 16, 16)] = jnp.broadcast_to(off_a, (16,))
            return off_a + cntv[0]

        off_end = lax.fori_loop(0, GIDX // 16, pass_a, off)

        def pass_b(k, z):
            d16 = gdst_v[pl.ds(base + k * 16, 16)] - row0
            s16 = gsrc_v[pl.ds(base + k * 16, 16)]
            m = (d16 >= 0) & (d16 < OWN)
            mi = jnp.where(m, jnp.full((16,), 1, jnp.int32),
                           jnp.full((16,), 0, jnp.int32))
            cs = plsc.cumsum(mi)
            pv = pbuf_v[pl.ds(k * 16, 16)]
            pos = jnp.where(m, (pv + cs - 1) & (SBUF - 1), SBUF + lane)
            plsc.store_scatter(cdst_v, [pos], d16 + win0)
            plsc.store_scatter(csrc_v, [pos], s16)
            return z

        lax.fori_loop(0, GIDX // 16, pass_b, 0)

        nf = flush_blocks(off_end // FLUSH - nf, nf)
        return off_end, nf

    off, nf = lax.fori_loop(0, NGRP, group_body,
                            (jnp.int32(0), jnp.int32(0)))

    npadv = (FLUSH - lax.rem(off, FLUSH) + 15) // 16

    def pad_body(i, off3):
        pos = (off3 + lane) & (SBUF - 1)
        plsc.store_scatter(csrc_v, [pos], pad_src)
        plsc.store_scatter(cdst_v, [pos], pad_dst)
        return off3 + 16

    off = lax.fori_loop(0, npadv, pad_body, off)
    nf = flush_blocks(off // FLUSH - nf, nf)

    tmp_v[pl.ds(0, 16)] = jnp.broadcast_to(nf, (16,))
    pltpu.sync_copy(tmp_v, cnt_hbm.at[w])


def _sc_bucketize(srcg, dstg):
    mesh = plsc.VectorSubcoreMesh(core_axis_name="c", subcore_axis_name="s")
    return pl.kernel(
        _sc_bucketize_kernel,
        out_type=(
            jax.ShapeDtypeStruct((NW, CAP), jnp.int32),
            jax.ShapeDtypeStruct((NW, CAP), jnp.int32),
            jax.ShapeDtypeStruct((NW, 16), jnp.int32),
        ),
        mesh=mesh,
        compiler_params=pltpu.CompilerParams(needs_layout_passes=False),
        scratch_types=[
            pltpu.VMEM((2 * GIDX,), jnp.int32),
            pltpu.VMEM((2 * GIDX,), jnp.int32),
            pltpu.VMEM((SBUF + 16,), jnp.int32),
            pltpu.VMEM((SBUF + 16,), jnp.int32),
            pltpu.VMEM((GIDX,), jnp.int32),
            pltpu.VMEM((16,), jnp.int32),
            pltpu.SemaphoreType.DMA,
            pltpu.SemaphoreType.DMA,
        ],
    )(srcg, dstg)


def _sc_agg_kernel(x_hbm, lsrc_hbm, ldst_hbm, cnt_hbm, out_hbm,
                   sidx_v, didx_v, rows_a, rows_b, zbuf_v, tmp_v, acc_sh,
                   g0, g1, s0, s1):
    c = lax.axis_index("c")
    s = lax.axis_index("s")
    w = c * NS + s
    win0 = s * ACC_ROWS

    zeros16 = jnp.zeros((16,), jnp.float32)

    def zrow(i, carry):
        for j in range(D // 16):
            zbuf_v[i, pl.ds(j * 16, 16)] = zeros16
        return carry

    lax.fori_loop(0, ACC_ROWS, zrow, 0)

    pltpu.sync_copy(cnt_hbm.at[w], tmp_v)
    nch = tmp_v[pl.ds(0, 16)][0] * (FLUSH // CHUNK)

    pltpu.sync_copy(zbuf_v, acc_sh.at[pl.ds(win0, ACC_ROWS)])

    def idx_dma(ci, slot):
        pltpu.sync_copy(lsrc_hbm.at[w, pl.ds(ci * CHUNK, CHUNK)],
                        sidx_v.at[slot])
        pltpu.sync_copy(ldst_hbm.at[w, pl.ds(ci * CHUNK, CHUNK)],
                        didx_v.at[slot])

    @pl.when(nch > 0)
    def _():
        idx_dma(0, 0)
        pltpu.async_copy(x_hbm.at[sidx_v.at[0]], rows_a, g0)

    @pl.when(nch > 1)
    def _():
        idx_dma(1, 1)

    def chunk_body(ci, carry):
        b = lax.rem(ci, 2)
        q = lax.rem(ci, 4)
        qn = lax.rem(ci + 1, 4)
        qp = lax.rem(ci + 2, 4)

        @pl.when(b == 0)
        def _():
            pltpu.make_async_copy(x_hbm.at[sidx_v.at[q]], rows_a, g0).wait()

            @pl.when(ci >= 1)
            def _():
                pltpu.make_async_copy(rows_b, acc_sh.at[didx_v.at[0]],
                                      s1).wait()

            @pl.when(ci + 1 < nch)
            def _():
                pltpu.async_copy(x_hbm.at[sidx_v.at[qn]], rows_b, g1)
            pltpu.async_copy(rows_a, acc_sh.at[didx_v.at[q]], s0, add=True)

            @pl.when(ci + 2 < nch)
            def _():
                idx_dma(ci + 2, qp)

        @pl.when(b == 1)
        def _():
            pltpu.make_async_copy(x_hbm.at[sidx_v.at[q]], rows_b, g1).wait()
            pltpu.make_async_copy(rows_a, acc_sh.at[didx_v.at[0]], s0).wait()

            @pl.when(ci + 1 < nch)
            def _():
                pltpu.async_copy(x_hbm.at[sidx_v.at[qn]], rows_a, g0)
            pltpu.async_copy(rows_b, acc_sh.at[didx_v.at[q]], s1, add=True)

            @pl.when(ci + 2 < nch)
            def _():
                idx_dma(ci + 2, qp)

        return carry

    lax.fori_loop(0, nch, chunk_body, 0)

    @pl.when(nch > 0)
    def _():
        @pl.when(lax.rem(nch - 1, 2) == 0)
        def _():
            pltpu.make_async_copy(rows_a, acc_sh.at[didx_v.at[0]],
                                  s0).wait()

        @pl.when(lax.rem(nch - 1, 2) == 1)
        def _():
            pltpu.make_async_copy(rows_b, acc_sh.at[didx_v.at[0]],
                                  s1).wait()

    pltpu.sync_copy(acc_sh.at[pl.ds(win0, OWN)],
                    out_hbm.at[pl.ds(w * OWN, OWN)])


def _sc_agg(x, lsrc, ldst, cnt):
    mesh = plsc.VectorSubcoreMesh(core_axis_name="c", subcore_axis_name="s")
    return pl.kernel(
        _sc_agg_kernel,
        out_type=jax.ShapeDtypeStruct((OUT_ROWS, D), jnp.float32),
        mesh=mesh,
        scratch_types=[
            pltpu.VMEM((4, CHUNK), jnp.int32),
            pltpu.VMEM((4, CHUNK), jnp.int32),
            pltpu.VMEM((CHUNK, D), jnp.float32),
            pltpu.VMEM((CHUNK, D), jnp.float32),
            pltpu.VMEM((ACC_ROWS, D), jnp.float32),
            pltpu.VMEM((16,), jnp.int32),
            pltpu.VMEM_SHARED((NS * ACC_ROWS, D), jnp.float32),
            pltpu.SemaphoreType.DMA,
            pltpu.SemaphoreType.DMA,
            pltpu.SemaphoreType.DMA,
            pltpu.SemaphoreType.DMA,
        ],
    )(x, lsrc, ldst, cnt)


BLK = 1000


def _dense_core(x_ref, a_ref, w0_ref, w1_ref, b_ref):
    y = lax.dot_general(x_ref[...], w0_ref[...], (((1,), (0,)), ((), ())),
                        preferred_element_type=jnp.float32)
    y = y + lax.dot_general(a_ref[...], w1_ref[...], (((1,), (0,)), ((), ())),
                            preferred_element_type=jnp.float32)
    return y + b_ref[...]


def _dense_body_ln(x_ref, a_ref, w0_ref, w1_ref, b_ref, o_ref):
    y = _dense_core(x_ref, a_ref, w0_ref, w1_ref, b_ref)
    m = jnp.mean(y, axis=1, keepdims=True)
    v = jnp.mean(jnp.square(y - m), axis=1, keepdims=True)
    y = (y - m) * lax.rsqrt(v + 1e-5)
    o_ref[...] = jnp.maximum(y, 0.0)


def _dense_body_ln_res(x_ref, a_ref, w0_ref, w1_ref, b_ref, r_ref, o_ref):
    y = _dense_core(x_ref, a_ref, w0_ref, w1_ref, b_ref)
    m = jnp.mean(y, axis=1, keepdims=True)
    v = jnp.mean(jnp.square(y - m), axis=1, keepdims=True)
    y = (y - m) * lax.rsqrt(v + 1e-5)
    o_ref[...] = jnp.maximum(y, 0.0) + r_ref[...]


def _dense_body_plain(x_ref, a_ref, w0_ref, w1_ref, b_ref, o_ref):
    o_ref[...] = _dense_core(x_ref, a_ref, w0_ref, w1_ref, b_ref)


def _dense(x, agg, w0, w1, b, res=None, ln_relu=True):
    grid = (N // BLK,)
    in_specs = [
        pl.BlockSpec((BLK, D), lambda i: (i, 0)),
        pl.BlockSpec((BLK, D), lambda i: (i, 0)),
        pl.BlockSpec((D, D), lambda i: (0, 0)),
        pl.BlockSpec((D, D), lambda i: (0, 0)),
        pl.BlockSpec((1, D), lambda i: (0, 0)),
    ]
    args = [x, agg, w0, w1, b]
    if res is not None:
        body = _dense_body_ln_res
        in_specs.append(pl.BlockSpec((BLK, D), lambda i: (i, 0)))
        args.append(res)
    elif ln_relu:
        body = _dense_body_ln
    else:
        body = _dense_body_plain
    return pl.pallas_call(
        body,
        grid=grid,
        in_specs=in_specs,
        out_specs=pl.BlockSpec((BLK, D), lambda i: (i, 0)),
        out_shape=jax.ShapeDtypeStruct((N, D), jnp.float32),
    )(*args)


def _stage(x):
    return pl.pallas_call(
        lambda i_ref, o_ref: o_ref.__setitem__((...,), i_ref[...]),
        grid=(N // BLK,),
        in_specs=[pl.BlockSpec((BLK, D), lambda i: (i, 0))],
        out_specs=pl.BlockSpec((BLK, D), lambda i: (i, 0)),
        out_shape=jax.ShapeDtypeStruct((N, D), jnp.float32),
    )(x)


def kernel(features, edges, w0_in, w1_in, b_in, w0_h1, w1_h1, b_h1,
           w0_h2, w1_h2, b_h2, w0_out, w1_out, b_out):
    srcg = jnp.concatenate([edges[:, 0], edges[:, 1]])
    dstg = jnp.concatenate([edges[:, 1], edges[:, 0]])

    lsrc, ldst, cnt = _sc_bucketize(srcg, dstg)

    b_in2 = b_in.reshape(1, -1)
    b_h12 = b_h1.reshape(1, -1)
    b_h22 = b_h2.reshape(1, -1)
    w0o = jnp.zeros((D, D), jnp.float32).at[:, :NDIMS].set(w0_out)
    w1o = jnp.zeros((D, D), jnp.float32).at[:, :NDIMS].set(w1_out)
    bo = jnp.zeros((1, D), jnp.float32).at[0, :NDIMS].set(b_out)

    a = _sc_agg(_stage(features), lsrc, ldst, cnt)
    h0 = _dense(features, a, w0_in, w1_in, b_in2)
    a = _sc_agg(_stage(h0), lsrc, ldst, cnt)
    h1 = _dense(h0, a, w0_h1, w1_h1, b_h12)
    a = _sc_agg(_stage(h1), lsrc, ldst, cnt)
    latent = _dense(h1, a, w0_h2, w1_h2, b_h22, res=h0)
    a = _sc_agg(_stage(latent), lsrc, ldst, cnt)
    out = _dense(latent, a, w0o, w1o, bo, ln_relu=False)
    return out[:, :NDIMS].reshape(1, 1, N, NDIMS)
